# Initial kernel scaffold; baseline (speedup 1.0000x reference)
import jax
import jax.numpy as jnp
from jax.experimental import pallas as pl


def kernel(y, x, Wq, bq, Wk, bk, Woff, boff):
    raise NotImplementedError("write your pallas kernel here")


# speedup vs baseline: 1611.4747x; 1611.4747x over previous
import functools

import jax
import jax.numpy as jnp
from jax import lax
from jax.experimental import pallas as pl
from jax.experimental.pallas import tpu as pltpu
from jax.experimental.pallas import tpu_sc as plsc

UP = 2
P = 9
G = 4
E = 32
DG = 64
H = 64
W = 64
HU = H * UP
WU = W * UP
NPIX = HU * WU
NSITE = H * W

NC = 2
NS = 16
NW = NC * NS
CH = 128
PIX_PER_W = 2 * G * NPIX // NW
NCHUNK = PIX_PER_W // CH


def _mm_body(x_ref, w_ref, b_ref, o_ref):
    o_ref[...] = (
        jnp.dot(w_ref[...], x_ref[...], preferred_element_type=jnp.float32)
        + b_ref[...]
    )


def _conv_tc(x2, Wc, bc, tn):
    B, C, S = x2.shape
    OC = Wc.shape[0]
    grid = (B, S // tn)
    return pl.pallas_call(
        _mm_body,
        grid=grid,
        in_specs=[
            pl.BlockSpec((None, C, tn), lambda b, t: (b, 0, t)),
            pl.BlockSpec((OC, C), lambda b, t: (0, 0)),
            pl.BlockSpec((OC, 1), lambda b, t: (0, 0)),
        ],
        out_specs=pl.BlockSpec((None, OC, tn), lambda b, t: (b, 0, t)),
        out_shape=jax.ShapeDtypeStruct((B, OC, S), jnp.float32),
    )(x2, Wc, bc.reshape(OC, 1))


def _sc_body(qs_hbm, off_hbm, kt_hbm, vt_hbm, out_hbm,
             off_buf, q_buf, idx_buf, w_buf, krows, vrows, attn, out_buf, sem):
    wid = lax.axis_index("s") * NC + lax.axis_index("c")
    bg = wid // 4
    part = wid - bg * 4
    t0 = part * PIX_PER_W
    iota = lax.iota(jnp.int32, 16)

    def chunk_body(ci, _):
        t = t0 + ci * CH
        pltpu.sync_copy(off_hbm.at[bg, :, pl.ds(t, CH)], off_buf)
        pltpu.sync_copy(qs_hbm.at[bg, :, pl.ds(t, CH)], q_buf)

        for p in range(P):
            pi = float(p // 3 - 1)
            pj = float(p % 3 - 1)

            def idx_grp(j, _, p=p, pi=pi, pj=pj):
                lanes = j * 16
                tv = t + lanes + iota
                hu = tv >> 7
                wu = tv & 127
                offy = off_buf[2 * p, pl.ds(lanes, 16)]
                offx = off_buf[2 * p + 1, pl.ds(lanes, 16)]
                cy = (hu.astype(jnp.float32) + 0.5) * 0.5 - 0.5 + pi + offy
                cx = (wu.astype(jnp.float32) + 0.5) * 0.5 - 0.5 + pj + offx
                yt = cy.astype(jnp.int32)
                y0 = jnp.where(yt.astype(jnp.float32) > cy, yt - 1, yt)
                xt = cx.astype(jnp.int32)
                x0 = jnp.where(xt.astype(jnp.float32) > cx, xt - 1, xt)
                wy = cy - y0.astype(jnp.float32)
                wx = cx - x0.astype(jnp.float32)
                for n, (dy, dx) in enumerate(((0, 0), (0, 1), (1, 0), (1, 1))):
                    yi = y0 + dy
                    xi = x0 + dx
                    ok = (yi >= 0) & (yi <= H - 1) & (xi >= 0) & (xi <= W - 1)
                    wn = (wy if dy else (1.0 - wy)) * (wx if dx else (1.0 - wx))
                    wn = jnp.where(ok, wn, 0.0)
                    yc = jnp.clip(yi, 0, H - 1)
                    xc = jnp.clip(xi, 0, W - 1)
                    idx_buf[p, n, pl.ds(lanes, 16)] = bg * NSITE + yc * W + xc
                    w_buf[p, n, pl.ds(lanes, 16)] = wn
                return 0

            lax.fori_loop(0, CH // 16, idx_grp, 0)
            descs = [
                pltpu.async_copy(kt_hbm.at[idx_buf.at[p, n]], krows.at[pl.ds(n * CH, CH)], sem)
                for n in range(4)
            ]
            for d in descs:
                d.wait()

            def dot_grp(j, _, p=p):
                lanes = j * 16
                sv = lanes + iota
                w0 = w_buf[p, 0, pl.ds(lanes, 16)]
                w1 = w_buf[p, 1, pl.ds(lanes, 16)]
                w2 = w_buf[p, 2, pl.ds(lanes, 16)]
                w3 = w_buf[p, 3, pl.ds(lanes, 16)]

                def e_step(e, acc):
                    ev = jnp.full((16,), e, jnp.int32)
                    kp = (w0 * plsc.load_gather(krows, [sv, ev])
                          + w1 * plsc.load_gather(krows, [sv + CH, ev])
                          + w2 * plsc.load_gather(krows, [sv + 2 * CH, ev])
                          + w3 * plsc.load_gather(krows, [sv + 3 * CH, ev]))
                    qv = q_buf[e, pl.ds(lanes, 16)]
                    return acc + qv * kp

                acc = lax.fori_loop(0, E, e_step, jnp.zeros((16,), jnp.float32))
                attn[p, pl.ds(lanes, 16)] = acc
                return 0

            lax.fori_loop(0, CH // 16, dot_grp, 0)

        def smax_grp(j, _):
            lanes = j * 16
            ls = [attn[p, pl.ds(lanes, 16)] for p in range(P)]
            m = ls[0]
            for p in range(1, P):
                m = jnp.maximum(m, ls[p])
            es = [jnp.exp(l - m) for l in ls]
            s = es[0]
            for p in range(1, P):
                s = s + es[p]
            inv = 1.0 / s
            for p in range(P):
                attn[p, pl.ds(lanes, 16)] = es[p] * inv
            return 0

        lax.fori_loop(0, CH // 16, smax_grp, 0)

        for p in range(P):
            descs = [
                pltpu.async_copy(vt_hbm.at[idx_buf.at[p, n]], vrows.at[pl.ds(n * CH, CH)], sem)
                for n in range(4)
            ]
            for d in descs:
                d.wait()

            def v_grp(j, _, p=p):
                lanes = j * 16
                sv = lanes + iota
                a = attn[p, pl.ds(lanes, 16)]
                w0 = a * w_buf[p, 0, pl.ds(lanes, 16)]
                w1 = a * w_buf[p, 1, pl.ds(lanes, 16)]
                w2 = a * w_buf[p, 2, pl.ds(lanes, 16)]
                w3 = a * w_buf[p, 3, pl.ds(lanes, 16)]

                def e_step(e, _):
                    ev = jnp.full((16,), e, jnp.int32)
                    v = (w0 * plsc.load_gather(vrows, [sv, ev])
                         + w1 * plsc.load_gather(vrows, [sv + CH, ev])
                         + w2 * plsc.load_gather(vrows, [sv + 2 * CH, ev])
                         + w3 * plsc.load_gather(vrows, [sv + 3 * CH, ev]))
                    if p == 0:
                        out_buf[e, pl.ds(lanes, 16)] = v
                    else:
                        out_buf[e, pl.ds(lanes, 16)] = out_buf[e, pl.ds(lanes, 16)] + v
                    return 0

                lax.fori_loop(0, DG, e_step, 0)
                return 0

            lax.fori_loop(0, CH // 16, v_grp, 0)

        pltpu.sync_copy(out_buf, out_hbm.at[bg, :, pl.ds(t, CH)])
        return 0

    lax.fori_loop(0, NCHUNK, chunk_body, 0)


_sc_call = functools.partial(
    pl.kernel,
    out_type=jax.ShapeDtypeStruct((2 * G, DG, NPIX), jnp.float32),
    mesh=plsc.VectorSubcoreMesh(core_axis_name="c", subcore_axis_name="s"),
    compiler_params=pltpu.CompilerParams(
        needs_layout_passes=False, use_tc_tiling_on_sc=False
    ),
    scratch_types=[
        pltpu.VMEM((2 * P, CH), jnp.float32),
        pltpu.VMEM((E, CH), jnp.float32),
        pltpu.VMEM((P, 4, CH), jnp.int32),
        pltpu.VMEM((P, 4, CH), jnp.float32),
        pltpu.VMEM((4 * CH, E), jnp.float32),
        pltpu.VMEM((4 * CH, DG), jnp.float32),
        pltpu.VMEM((P, CH), jnp.float32),
        pltpu.VMEM((DG, CH), jnp.float32),
        pltpu.SemaphoreType.DMA,
    ],
)(_sc_body)


def kernel(y, x, Wq, bq, Wk, bk, Woff, boff):
    B = y.shape[0]
    y2 = y.reshape(B, y.shape[1], NPIX)
    x2 = x.reshape(B, x.shape[1], NSITE)

    q = _conv_tc(y2, Wq, bq, 2048)
    Wkx = jnp.concatenate([Wk, Woff], axis=0)
    bkx = jnp.concatenate([bk, boff], axis=0)
    kx = _conv_tc(x2, Wkx, bkx, 2048)

    k_nat = kx[:, :G * E]
    off_raw = kx[:, G * E:]

    qs = q.reshape(B * G, E, NPIX)
    kt = k_nat.reshape(B, G, E, NSITE).transpose(0, 1, 3, 2).reshape(B * G * NSITE, E)
    vt = x2.reshape(B, G, DG, NSITE).transpose(0, 1, 3, 2).reshape(B * G * NSITE, DG)
    t_ = off_raw.reshape(B, G * P * 2, UP, UP, H, W)
    t_ = t_.transpose(0, 1, 4, 2, 5, 3).reshape(B, G * P * 2, HU, WU)
    off_s = t_.reshape(B * G, P * 2, NPIX)

    out = _sc_call(qs, off_s, kt, vt)
    return out.reshape(B, G * DG, HU, WU)


# speedup vs baseline: 1653.9338x; 1.0263x over previous
import functools

import jax
import jax.numpy as jnp
from jax import lax
from jax.experimental import pallas as pl
from jax.experimental.pallas import tpu as pltpu
from jax.experimental.pallas import tpu_sc as plsc

UP = 2
P = 9
G = 4
E = 32
DG = 64
H = 64
W = 64
HU = H * UP
WU = W * UP
NPIX = HU * WU
NSITE = H * W

NC = 2
NS = 16
NW = NC * NS
CH = 128
PIX_PER_W = 2 * G * NPIX // NW
NCHUNK = PIX_PER_W // CH


def _mm_body(x_ref, w_ref, b_ref, o_ref):
    o_ref[...] = (
        jnp.dot(w_ref[...], x_ref[...], preferred_element_type=jnp.float32)
        + b_ref[...]
    )


def _conv_tc(x2, Wc, bc, tn):
    B, C, S = x2.shape
    OC = Wc.shape[0]
    grid = (B, S // tn)
    return pl.pallas_call(
        _mm_body,
        grid=grid,
        in_specs=[
            pl.BlockSpec((None, C, tn), lambda b, t: (b, 0, t)),
            pl.BlockSpec((OC, C), lambda b, t: (0, 0)),
            pl.BlockSpec((OC, 1), lambda b, t: (0, 0)),
        ],
        out_specs=pl.BlockSpec((None, OC, tn), lambda b, t: (b, 0, t)),
        out_shape=jax.ShapeDtypeStruct((B, OC, S), jnp.float32),
    )(x2, Wc, bc.reshape(OC, 1))


def _sc_body(qs_hbm, off_hbm, kt_hbm, vt_hbm, out_hbm,
             off_buf, q_buf, idx_buf, w_buf, krows, vrows, attn, out_buf, sem):
    wid = lax.axis_index("s") * NC + lax.axis_index("c")
    bg = wid // 4
    part = wid - bg * 4
    t0 = part * PIX_PER_W
    iota = lax.iota(jnp.int32, 16)

    def chunk_body(ci, _):
        t = t0 + ci * CH
        pltpu.sync_copy(off_hbm.at[bg, :, pl.ds(t, CH)], off_buf)
        pltpu.sync_copy(qs_hbm.at[bg, :, pl.ds(t, CH)], q_buf)

        for p in range(P):
            pi = float(p // 3 - 1)
            pj = float(p % 3 - 1)

            def idx_grp(j, _, p=p, pi=pi, pj=pj):
                lanes = j * 16
                tv = t + lanes + iota
                hu = tv >> 7
                wu = tv & 127
                offy = off_buf[2 * p, pl.ds(lanes, 16)]
                offx = off_buf[2 * p + 1, pl.ds(lanes, 16)]
                cy = (hu.astype(jnp.float32) + 0.5) * 0.5 - 0.5 + pi + offy
                cx = (wu.astype(jnp.float32) + 0.5) * 0.5 - 0.5 + pj + offx
                yt = cy.astype(jnp.int32)
                y0 = jnp.where(yt.astype(jnp.float32) > cy, yt - 1, yt)
                xt = cx.astype(jnp.int32)
                x0 = jnp.where(xt.astype(jnp.float32) > cx, xt - 1, xt)
                wy = cy - y0.astype(jnp.float32)
                wx = cx - x0.astype(jnp.float32)
                for n, (dy, dx) in enumerate(((0, 0), (0, 1), (1, 0), (1, 1))):
                    yi = y0 + dy
                    xi = x0 + dx
                    ok = (yi >= 0) & (yi <= H - 1) & (xi >= 0) & (xi <= W - 1)
                    wn = (wy if dy else (1.0 - wy)) * (wx if dx else (1.0 - wx))
                    wn = jnp.where(ok, wn, 0.0)
                    yc = jnp.clip(yi, 0, H - 1)
                    xc = jnp.clip(xi, 0, W - 1)
                    idx_buf[p, n, pl.ds(lanes, 16)] = bg * NSITE + yc * W + xc
                    w_buf[p, n, pl.ds(lanes, 16)] = wn
                return 0

            lax.fori_loop(0, CH // 16, idx_grp, 0)
            descs = [
                pltpu.async_copy(kt_hbm.at[idx_buf.at[p, n]], krows.at[pl.ds(n * CH, CH)], sem)
                for n in range(4)
            ]
            for d in descs:
                d.wait()

            def dot_grp(j, _, p=p):
                lanes = j * 16
                sv = lanes + iota
                w0 = w_buf[p, 0, pl.ds(lanes, 16)]
                w1 = w_buf[p, 1, pl.ds(lanes, 16)]
                w2 = w_buf[p, 2, pl.ds(lanes, 16)]
                w3 = w_buf[p, 3, pl.ds(lanes, 16)]

                def e_step(e4, acc):
                    e = e4 * 4
                    for u in range(4):
                        ev = jnp.full((16,), e + u, jnp.int32)
                        kp = (w0 * plsc.load_gather(krows, [sv, ev])
                              + w1 * plsc.load_gather(krows, [sv + CH, ev])
                              + w2 * plsc.load_gather(krows, [sv + 2 * CH, ev])
                              + w3 * plsc.load_gather(krows, [sv + 3 * CH, ev]))
                        qv = q_buf[e + u, pl.ds(lanes, 16)]
                        acc = acc + qv * kp
                    return acc

                acc = lax.fori_loop(0, E // 4, e_step, jnp.zeros((16,), jnp.float32))
                attn[p, pl.ds(lanes, 16)] = acc
                return 0

            lax.fori_loop(0, CH // 16, dot_grp, 0)

        def smax_grp(j, _):
            lanes = j * 16
            ls = [attn[p, pl.ds(lanes, 16)] for p in range(P)]
            m = ls[0]
            for p in range(1, P):
                m = jnp.maximum(m, ls[p])
            es = [jnp.exp(l - m) for l in ls]
            s = es[0]
            for p in range(1, P):
                s = s + es[p]
            inv = 1.0 / s
            for p in range(P):
                attn[p, pl.ds(lanes, 16)] = es[p] * inv
            return 0

        lax.fori_loop(0, CH // 16, smax_grp, 0)

        for p in range(P):
            descs = [
                pltpu.async_copy(vt_hbm.at[idx_buf.at[p, n]], vrows.at[pl.ds(n * CH, CH)], sem)
                for n in range(4)
            ]
            for d in descs:
                d.wait()

            def v_grp(j, _, p=p):
                lanes = j * 16
                sv = lanes + iota
                a = attn[p, pl.ds(lanes, 16)]
                w0 = a * w_buf[p, 0, pl.ds(lanes, 16)]
                w1 = a * w_buf[p, 1, pl.ds(lanes, 16)]
                w2 = a * w_buf[p, 2, pl.ds(lanes, 16)]
                w3 = a * w_buf[p, 3, pl.ds(lanes, 16)]

                def e_step(e4, _):
                    e = e4 * 4
                    for u in range(4):
                        ev = jnp.full((16,), e + u, jnp.int32)
                        v = (w0 * plsc.load_gather(vrows, [sv, ev])
                             + w1 * plsc.load_gather(vrows, [sv + CH, ev])
                             + w2 * plsc.load_gather(vrows, [sv + 2 * CH, ev])
                             + w3 * plsc.load_gather(vrows, [sv + 3 * CH, ev]))
                        if p == 0:
                            out_buf[e + u, pl.ds(lanes, 16)] = v
                        else:
                            out_buf[e + u, pl.ds(lanes, 16)] = (
                                out_buf[e + u, pl.ds(lanes, 16)] + v)
                    return 0

                lax.fori_loop(0, DG // 4, e_step, 0)
                return 0

            lax.fori_loop(0, CH // 16, v_grp, 0)

        pltpu.sync_copy(out_buf, out_hbm.at[bg, :, pl.ds(t, CH)])
        return 0

    lax.fori_loop(0, NCHUNK, chunk_body, 0)


_sc_call = functools.partial(
    pl.kernel,
    out_type=jax.ShapeDtypeStruct((2 * G, DG, NPIX), jnp.float32),
    mesh=plsc.VectorSubcoreMesh(core_axis_name="c", subcore_axis_name="s"),
    compiler_params=pltpu.CompilerParams(
        needs_layout_passes=False, use_tc_tiling_on_sc=False
    ),
    scratch_types=[
        pltpu.VMEM((2 * P, CH), jnp.float32),
        pltpu.VMEM((E, CH), jnp.float32),
        pltpu.VMEM((P, 4, CH), jnp.int32),
        pltpu.VMEM((P, 4, CH), jnp.float32),
        pltpu.VMEM((4 * CH, E), jnp.float32),
        pltpu.VMEM((4 * CH, DG), jnp.float32),
        pltpu.VMEM((P, CH), jnp.float32),
        pltpu.VMEM((DG, CH), jnp.float32),
        pltpu.SemaphoreType.DMA,
    ],
)(_sc_body)


def kernel(y, x, Wq, bq, Wk, bk, Woff, boff):
    B = y.shape[0]
    y2 = y.reshape(B, y.shape[1], NPIX)
    x2 = x.reshape(B, x.shape[1], NSITE)

    q = _conv_tc(y2, Wq, bq, 2048)
    Wkx = jnp.concatenate([Wk, Woff], axis=0)
    bkx = jnp.concatenate([bk, boff], axis=0)
    kx = _conv_tc(x2, Wkx, bkx, 2048)

    k_nat = kx[:, :G * E]
    off_raw = kx[:, G * E:]

    qs = q.reshape(B * G, E, NPIX)
    kt = k_nat.reshape(B, G, E, NSITE).transpose(0, 1, 3, 2).reshape(B * G * NSITE, E)
    vt = x2.reshape(B, G, DG, NSITE).transpose(0, 1, 3, 2).reshape(B * G * NSITE, DG)
    t_ = off_raw.reshape(B, G * P * 2, UP, UP, H, W)
    t_ = t_.transpose(0, 1, 4, 2, 5, 3).reshape(B, G * P * 2, HU, WU)
    off_s = t_.reshape(B * G, P * 2, NPIX)

    out = _sc_call(qs, off_s, kt, vt)
    return out.reshape(B, G * DG, HU, WU)


# speedup vs baseline: 5289.3215x; 3.1980x over previous
import functools

import jax
import jax.numpy as jnp
from jax import lax
from jax.experimental import pallas as pl
from jax.experimental.pallas import tpu as pltpu
from jax.experimental.pallas import tpu_sc as plsc

UP = 2
P = 9
G = 4
E = 32
DG = 64
H = 64
W = 64
HU = H * UP
WU = W * UP
NPIX = HU * WU
NSITE = H * W

NC = 2
NS = 16
NW = NC * NS
CH = 128
PIX_PER_W = 2 * G * NPIX // NW
NCHUNK = PIX_PER_W // CH


def _mm_body(x_ref, w_ref, b_ref, o_ref):
    o_ref[...] = (
        jnp.dot(w_ref[...], x_ref[...], preferred_element_type=jnp.float32)
        + b_ref[...]
    )


def _conv_tc(x2, Wc, bc, tn):
    B, C, S = x2.shape
    OC = Wc.shape[0]
    grid = (B, S // tn)
    return pl.pallas_call(
        _mm_body,
        grid=grid,
        in_specs=[
            pl.BlockSpec((None, C, tn), lambda b, t: (b, 0, t)),
            pl.BlockSpec((OC, C), lambda b, t: (0, 0)),
            pl.BlockSpec((OC, 1), lambda b, t: (0, 0)),
        ],
        out_specs=pl.BlockSpec((None, OC, tn), lambda b, t: (b, 0, t)),
        out_shape=jax.ShapeDtypeStruct((B, OC, S), jnp.float32),
    )(x2, Wc, bc.reshape(OC, 1))


def _sc_body(qs_hbm, off_hbm, kt_hbm, vt_hbm, out_hbm,
             off_buf, q_buf, idx_buf, w_buf, krows, vrows, attn, out_buf, sem):
    wid = lax.axis_index("s") * NC + lax.axis_index("c")
    bg = wid // 4
    part = wid - bg * 4
    t0 = part * PIX_PER_W
    iota = lax.iota(jnp.int32, 16)

    def chunk_body(ci, _):
        t = t0 + ci * CH
        pltpu.sync_copy(off_hbm.at[bg, :, pl.ds(t, CH)], off_buf)
        pltpu.sync_copy(qs_hbm.at[bg, pl.ds(t, CH)], q_buf)

        for p in range(P):
            pi = float(p // 3 - 1)
            pj = float(p % 3 - 1)

            def idx_grp(j, _, p=p, pi=pi, pj=pj):
                lanes = j * 16
                tv = t + lanes + iota
                hu = tv >> 7
                wu = tv & 127
                offy = off_buf[2 * p, pl.ds(lanes, 16)]
                offx = off_buf[2 * p + 1, pl.ds(lanes, 16)]
                cy = (hu.astype(jnp.float32) + 0.5) * 0.5 - 0.5 + pi + offy
                cx = (wu.astype(jnp.float32) + 0.5) * 0.5 - 0.5 + pj + offx
                yt = cy.astype(jnp.int32)
                y0 = jnp.where(yt.astype(jnp.float32) > cy, yt - 1, yt)
                xt = cx.astype(jnp.int32)
                x0 = jnp.where(xt.astype(jnp.float32) > cx, xt - 1, xt)
                wy = cy - y0.astype(jnp.float32)
                wx = cx - x0.astype(jnp.float32)
                for n, (dy, dx) in enumerate(((0, 0), (0, 1), (1, 0), (1, 1))):
                    yi = y0 + dy
                    xi = x0 + dx
                    ok = (yi >= 0) & (yi <= H - 1) & (xi >= 0) & (xi <= W - 1)
                    wn = (wy if dy else (1.0 - wy)) * (wx if dx else (1.0 - wx))
                    wn = jnp.where(ok, wn, 0.0)
                    yc = jnp.clip(yi, 0, H - 1)
                    xc = jnp.clip(xi, 0, W - 1)
                    idx_buf[p, n, pl.ds(lanes, 16)] = bg * NSITE + yc * W + xc
                    w_buf[p, n, pl.ds(lanes, 16)] = wn
                return 0

            lax.fori_loop(0, CH // 16, idx_grp, 0)
            descs = [
                pltpu.async_copy(kt_hbm.at[idx_buf.at[p, n]], krows.at[pl.ds(n * CH, CH)], sem)
                for n in range(4)
            ]
            for d in descs:
                d.wait()

            lane15 = iota == 15

            def dot_smp(s2, _, p=p):
                for u in range(2):
                    s = s2 * 2 + u
                    w0 = w_buf[p, 0, pl.ds(s, 16)][0]
                    w1 = w_buf[p, 1, pl.ds(s, 16)][0]
                    w2 = w_buf[p, 2, pl.ds(s, 16)][0]
                    w3 = w_buf[p, 3, pl.ds(s, 16)][0]
                    acc0 = (w0 * krows[s, pl.ds(0, 16)]
                            + w1 * krows[CH + s, pl.ds(0, 16)]
                            + w2 * krows[2 * CH + s, pl.ds(0, 16)]
                            + w3 * krows[3 * CH + s, pl.ds(0, 16)])
                    acc1 = (w0 * krows[s, pl.ds(16, 16)]
                            + w1 * krows[CH + s, pl.ds(16, 16)]
                            + w2 * krows[2 * CH + s, pl.ds(16, 16)]
                            + w3 * krows[3 * CH + s, pl.ds(16, 16)])
                    prod = (acc0 * q_buf[s, pl.ds(0, 16)]
                            + acc1 * q_buf[s, pl.ds(16, 16)])
                    csum = plsc.cumsum(prod)
                    plsc.store_scatter(attn.at[p],
                                       [jnp.full((16,), s, jnp.int32)],
                                       csum, mask=lane15)
                return 0

            lax.fori_loop(0, CH // 2, dot_smp, 0)

        def smax_grp(j, _):
            lanes = j * 16
            ls = [attn[p, pl.ds(lanes, 16)] for p in range(P)]
            m = ls[0]
            for p in range(1, P):
                m = jnp.maximum(m, ls[p])
            es = [jnp.exp(l - m) for l in ls]
            s = es[0]
            for p in range(1, P):
                s = s + es[p]
            inv = 1.0 / s
            for p in range(P):
                attn[p, pl.ds(lanes, 16)] = es[p] * inv
            return 0

        lax.fori_loop(0, CH // 16, smax_grp, 0)

        for p in range(P):
            descs = [
                pltpu.async_copy(vt_hbm.at[idx_buf.at[p, n]], vrows.at[pl.ds(n * CH, CH)], sem)
                for n in range(4)
            ]
            for d in descs:
                d.wait()

            def v_smp(s, _, p=p):
                a = attn[p, pl.ds(s, 16)][0]
                w0 = a * w_buf[p, 0, pl.ds(s, 16)][0]
                w1 = a * w_buf[p, 1, pl.ds(s, 16)][0]
                w2 = a * w_buf[p, 2, pl.ds(s, 16)][0]
                w3 = a * w_buf[p, 3, pl.ds(s, 16)][0]
                for h in range(DG // 16):
                    sl = pl.ds(h * 16, 16)
                    v = (w0 * vrows[s, sl]
                         + w1 * vrows[CH + s, sl]
                         + w2 * vrows[2 * CH + s, sl]
                         + w3 * vrows[3 * CH + s, sl])
                    if p == 0:
                        out_buf[s, sl] = v
                    else:
                        out_buf[s, sl] = out_buf[s, sl] + v
                return 0

            lax.fori_loop(0, CH, v_smp, 0)

        pltpu.sync_copy(out_buf, out_hbm.at[bg, pl.ds(t, CH)])
        return 0

    lax.fori_loop(0, NCHUNK, chunk_body, 0)


_sc_call = functools.partial(
    pl.kernel,
    out_type=jax.ShapeDtypeStruct((2 * G, NPIX, DG), jnp.float32),
    mesh=plsc.VectorSubcoreMesh(core_axis_name="c", subcore_axis_name="s"),
    compiler_params=pltpu.CompilerParams(
        needs_layout_passes=False, use_tc_tiling_on_sc=False
    ),
    scratch_types=[
        pltpu.VMEM((2 * P, CH), jnp.float32),
        pltpu.VMEM((CH, E), jnp.float32),
        pltpu.VMEM((P, 4, CH), jnp.int32),
        pltpu.VMEM((P, 4, CH + 16), jnp.float32),
        pltpu.VMEM((4 * CH, E), jnp.float32),
        pltpu.VMEM((4 * CH, DG), jnp.float32),
        pltpu.VMEM((P, CH + 16), jnp.float32),
        pltpu.VMEM((CH, DG), jnp.float32),
        pltpu.SemaphoreType.DMA,
    ],
)(_sc_body)


def kernel(y, x, Wq, bq, Wk, bk, Woff, boff):
    B = y.shape[0]
    y2 = y.reshape(B, y.shape[1], NPIX)
    x2 = x.reshape(B, x.shape[1], NSITE)

    q = _conv_tc(y2, Wq, bq, 2048)
    Wkx = jnp.concatenate([Wk, Woff], axis=0)
    bkx = jnp.concatenate([bk, boff], axis=0)
    kx = _conv_tc(x2, Wkx, bkx, 2048)

    k_nat = kx[:, :G * E]
    off_raw = kx[:, G * E:]

    qs = q.reshape(B, G, E, NPIX).transpose(0, 1, 3, 2).reshape(B * G, NPIX, E)
    kt = k_nat.reshape(B, G, E, NSITE).transpose(0, 1, 3, 2).reshape(B * G * NSITE, E)
    vt = x2.reshape(B, G, DG, NSITE).transpose(0, 1, 3, 2).reshape(B * G * NSITE, DG)
    t_ = off_raw.reshape(B, G * P * 2, UP, UP, H, W)
    t_ = t_.transpose(0, 1, 4, 2, 5, 3).reshape(B, G * P * 2, HU, WU)
    off_s = t_.reshape(B * G, P * 2, NPIX)

    out = _sc_call(qs, off_s, kt, vt)
    return out.transpose(0, 2, 1).reshape(B, G * DG, HU, WU)


# speedup vs baseline: 7102.7478x; 1.3428x over previous
import functools

import jax
import jax.numpy as jnp
from jax import lax
from jax.experimental import pallas as pl
from jax.experimental.pallas import tpu as pltpu
from jax.experimental.pallas import tpu_sc as plsc

UP = 2
P = 9
G = 4
E = 32
DG = 64
H = 64
W = 64
HU = H * UP
WU = W * UP
NPIX = HU * WU
NSITE = H * W

NC = 2
NS = 16
NW = NC * NS
CH = 128
PIX_PER_W = 2 * G * NPIX // NW
NCHUNK = PIX_PER_W // CH


def _mm_body(x_ref, w_ref, b_ref, o_ref):
    o_ref[...] = (
        jnp.dot(w_ref[...], x_ref[...], preferred_element_type=jnp.float32)
        + b_ref[...]
    )


def _conv_tc(x2, Wc, bc, tn):
    B, C, S = x2.shape
    OC = Wc.shape[0]
    grid = (B, S // tn)
    return pl.pallas_call(
        _mm_body,
        grid=grid,
        in_specs=[
            pl.BlockSpec((None, C, tn), lambda b, t: (b, 0, t)),
            pl.BlockSpec((OC, C), lambda b, t: (0, 0)),
            pl.BlockSpec((OC, 1), lambda b, t: (0, 0)),
        ],
        out_specs=pl.BlockSpec((None, OC, tn), lambda b, t: (b, 0, t)),
        out_shape=jax.ShapeDtypeStruct((B, OC, S), jnp.float32),
    )(x2, Wc, bc.reshape(OC, 1))


def _sc_body(qs_hbm, off_hbm, kt_hbm, vt_hbm, out_hbm,
             off_buf, q_buf, idx_buf, w_buf, krows_a, krows_b, vrows_a, vrows_b,
             attn, out_buf, sem_a, sem_b):
    wid = lax.axis_index("s") * NC + lax.axis_index("c")
    bg = wid // 4
    part = wid - bg * 4
    t0 = part * PIX_PER_W
    iota = lax.iota(jnp.int32, 16)
    kbufs = (krows_a, krows_b)
    vbufs = (vrows_a, vrows_b)
    sems = (sem_a, sem_b)

    def chunk_body(ci, _):
        t = t0 + ci * CH
        pltpu.sync_copy(off_hbm.at[bg, :, pl.ds(t, CH)], off_buf)
        pltpu.sync_copy(qs_hbm.at[bg, pl.ds(t, CH)], q_buf)

        for p in range(P):
            pi = float(p // 3 - 1)
            pj = float(p % 3 - 1)

            def idx_grp(j, _, p=p, pi=pi, pj=pj):
                lanes = j * 16
                tv = t + lanes + iota
                hu = tv >> 7
                wu = tv & 127
                offy = off_buf[2 * p, pl.ds(lanes, 16)]
                offx = off_buf[2 * p + 1, pl.ds(lanes, 16)]
                cy = (hu.astype(jnp.float32) + 0.5) * 0.5 - 0.5 + pi + offy
                cx = (wu.astype(jnp.float32) + 0.5) * 0.5 - 0.5 + pj + offx
                yt = cy.astype(jnp.int32)
                y0 = jnp.where(yt.astype(jnp.float32) > cy, yt - 1, yt)
                xt = cx.astype(jnp.int32)
                x0 = jnp.where(xt.astype(jnp.float32) > cx, xt - 1, xt)
                wy = cy - y0.astype(jnp.float32)
                wx = cx - x0.astype(jnp.float32)
                for n, (dy, dx) in enumerate(((0, 0), (0, 1), (1, 0), (1, 1))):
                    yi = y0 + dy
                    xi = x0 + dx
                    ok = (yi >= 0) & (yi <= H - 1) & (xi >= 0) & (xi <= W - 1)
                    wn = (wy if dy else (1.0 - wy)) * (wx if dx else (1.0 - wx))
                    wn = jnp.where(ok, wn, 0.0)
                    yc = jnp.clip(yi, 0, H - 1)
                    xc = jnp.clip(xi, 0, W - 1)
                    idx_buf[p, n, pl.ds(lanes, 16)] = bg * NSITE + yc * W + xc
                    w_buf[p, n, pl.ds(lanes, 16)] = wn
                return 0

            lax.fori_loop(0, CH // 16, idx_grp, 0)

        def issue(tab, p, buf, sem):
            return [
                pltpu.async_copy(tab.at[idx_buf.at[p, n]],
                                 buf.at[pl.ds(n * CH, CH)], sem)
                for n in range(4)
            ]

        lane15 = iota == 15

        descs = issue(kt_hbm, 0, kbufs[0], sems[0])
        for p in range(P):
            nxt = (issue(kt_hbm, p + 1, kbufs[(p + 1) % 2], sems[(p + 1) % 2])
                   if p + 1 < P else None)
            for d in descs:
                d.wait()
            descs = nxt
            krows = kbufs[p % 2]

            def dot_smp(s2, _, p=p, krows=krows):
                for u in range(2):
                    s = s2 * 2 + u
                    w0 = w_buf[p, 0, pl.ds(s, 16)][0]
                    w1 = w_buf[p, 1, pl.ds(s, 16)][0]
                    w2 = w_buf[p, 2, pl.ds(s, 16)][0]
                    w3 = w_buf[p, 3, pl.ds(s, 16)][0]
                    acc0 = (w0 * krows[s, pl.ds(0, 16)]
                            + w1 * krows[CH + s, pl.ds(0, 16)]
                            + w2 * krows[2 * CH + s, pl.ds(0, 16)]
                            + w3 * krows[3 * CH + s, pl.ds(0, 16)])
                    acc1 = (w0 * krows[s, pl.ds(16, 16)]
                            + w1 * krows[CH + s, pl.ds(16, 16)]
                            + w2 * krows[2 * CH + s, pl.ds(16, 16)]
                            + w3 * krows[3 * CH + s, pl.ds(16, 16)])
                    prod = (acc0 * q_buf[s, pl.ds(0, 16)]
                            + acc1 * q_buf[s, pl.ds(16, 16)])
                    csum = plsc.cumsum(prod)
                    plsc.store_scatter(attn.at[p],
                                       [jnp.full((16,), s, jnp.int32)],
                                       csum, mask=lane15)
                return 0

            lax.fori_loop(0, CH // 2, dot_smp, 0)

        def smax_grp(j, _):
            lanes = j * 16
            ls = [attn[p, pl.ds(lanes, 16)] for p in range(P)]
            m = ls[0]
            for p in range(1, P):
                m = jnp.maximum(m, ls[p])
            es = [jnp.exp(l - m) for l in ls]
            ssum = es[0]
            for p in range(1, P):
                ssum = ssum + es[p]
            inv = 1.0 / ssum
            for p in range(P):
                attn[p, pl.ds(lanes, 16)] = es[p] * inv
            return 0

        lax.fori_loop(0, CH // 16, smax_grp, 0)

        descs = issue(vt_hbm, 0, vbufs[0], sems[0])
        for p in range(P):
            nxt = (issue(vt_hbm, p + 1, vbufs[(p + 1) % 2], sems[(p + 1) % 2])
                   if p + 1 < P else None)
            for d in descs:
                d.wait()
            descs = nxt
            vrows = vbufs[p % 2]

            def v_smp(s, _, p=p, vrows=vrows):
                a = attn[p, pl.ds(s, 16)][0]
                w0 = a * w_buf[p, 0, pl.ds(s, 16)][0]
                w1 = a * w_buf[p, 1, pl.ds(s, 16)][0]
                w2 = a * w_buf[p, 2, pl.ds(s, 16)][0]
                w3 = a * w_buf[p, 3, pl.ds(s, 16)][0]
                for h in range(DG // 16):
                    sl = pl.ds(h * 16, 16)
                    v = (w0 * vrows[s, sl]
                         + w1 * vrows[CH + s, sl]
                         + w2 * vrows[2 * CH + s, sl]
                         + w3 * vrows[3 * CH + s, sl])
                    if p == 0:
                        out_buf[s, sl] = v
                    else:
                        out_buf[s, sl] = out_buf[s, sl] + v
                return 0

            lax.fori_loop(0, CH, v_smp, 0)

        pltpu.sync_copy(out_buf, out_hbm.at[bg, pl.ds(t, CH)])
        return 0

    lax.fori_loop(0, NCHUNK, chunk_body, 0)


_sc_call = functools.partial(
    pl.kernel,
    out_type=jax.ShapeDtypeStruct((2 * G, NPIX, DG), jnp.float32),
    mesh=plsc.VectorSubcoreMesh(core_axis_name="c", subcore_axis_name="s"),
    compiler_params=pltpu.CompilerParams(
        needs_layout_passes=False, use_tc_tiling_on_sc=False
    ),
    scratch_types=[
        pltpu.VMEM((2 * P, CH), jnp.float32),
        pltpu.VMEM((CH, E), jnp.float32),
        pltpu.VMEM((P, 4, CH), jnp.int32),
        pltpu.VMEM((P, 4, CH + 16), jnp.float32),
        pltpu.VMEM((4 * CH, E), jnp.float32),
        pltpu.VMEM((4 * CH, E), jnp.float32),
        pltpu.VMEM((4 * CH, DG), jnp.float32),
        pltpu.VMEM((4 * CH, DG), jnp.float32),
        pltpu.VMEM((P, CH + 16), jnp.float32),
        pltpu.VMEM((CH, DG), jnp.float32),
        pltpu.SemaphoreType.DMA,
        pltpu.SemaphoreType.DMA,
    ],
)(_sc_body)


def kernel(y, x, Wq, bq, Wk, bk, Woff, boff):
    B = y.shape[0]
    y2 = y.reshape(B, y.shape[1], NPIX)
    x2 = x.reshape(B, x.shape[1], NSITE)

    q = _conv_tc(y2, Wq, bq, 2048)
    Wkx = jnp.concatenate([Wk, Woff], axis=0)
    bkx = jnp.concatenate([bk, boff], axis=0)
    kx = _conv_tc(x2, Wkx, bkx, 2048)

    k_nat = kx[:, :G * E]
    off_raw = kx[:, G * E:]

    qs = q.reshape(B, G, E, NPIX).transpose(0, 1, 3, 2).reshape(B * G, NPIX, E)
    kt = k_nat.reshape(B, G, E, NSITE).transpose(0, 1, 3, 2).reshape(B * G * NSITE, E)
    vt = x2.reshape(B, G, DG, NSITE).transpose(0, 1, 3, 2).reshape(B * G * NSITE, DG)
    t_ = off_raw.reshape(B, G * P * 2, UP, UP, H, W)
    t_ = t_.transpose(0, 1, 4, 2, 5, 3).reshape(B, G * P * 2, HU, WU)
    off_s = t_.reshape(B * G, P * 2, NPIX)

    out = _sc_call(qs, off_s, kt, vt)
    return out.transpose(0, 2, 1).reshape(B, G * DG, HU, WU)


# speedup vs baseline: 8088.0454x; 1.1387x over previous
import functools

import jax
import jax.numpy as jnp
from jax import lax
from jax.experimental import pallas as pl
from jax.experimental.pallas import tpu as pltpu
from jax.experimental.pallas import tpu_sc as plsc

UP = 2
P = 9
G = 4
E = 32
DG = 64
H = 64
W = 64
HU = H * UP
WU = W * UP
NPIX = HU * WU
NSITE = H * W

NC = 2
NS = 16
NW = NC * NS
CH = 128
PIX_PER_W = 2 * G * NPIX // NW
NCHUNK = PIX_PER_W // CH


def _mm_body(x_ref, w_ref, b_ref, o_ref):
    o_ref[...] = (
        jnp.dot(w_ref[...], x_ref[...], preferred_element_type=jnp.float32)
        + b_ref[...]
    )


def _conv_tc(x2, Wc, bc, tn):
    B, C, S = x2.shape
    OC = Wc.shape[0]
    grid = (B, S // tn)
    return pl.pallas_call(
        _mm_body,
        grid=grid,
        in_specs=[
            pl.BlockSpec((None, C, tn), lambda b, t: (b, 0, t)),
            pl.BlockSpec((OC, C), lambda b, t: (0, 0)),
            pl.BlockSpec((OC, 1), lambda b, t: (0, 0)),
        ],
        out_specs=pl.BlockSpec((None, OC, tn), lambda b, t: (b, 0, t)),
        out_shape=jax.ShapeDtypeStruct((B, OC, S), jnp.float32),
    )(x2, Wc, bc.reshape(OC, 1))


def _sc_body(qs_hbm, off_hbm, kt_hbm, vt_hbm, out_hbm,
             off_buf, q_buf, idx_buf, w_buf, krows_a, krows_b, vrows_a, vrows_b,
             attn, out_buf, sem_a, sem_b):
    wid = lax.axis_index("s") * NC + lax.axis_index("c")
    bg = wid // 4
    part = wid - bg * 4
    t0 = part * PIX_PER_W
    iota = lax.iota(jnp.int32, 16)
    kbufs = (krows_a, krows_b)
    vbufs = (vrows_a, vrows_b)
    sems = (sem_a, sem_b)

    def chunk_body(ci, _):
        t = t0 + ci * CH
        pltpu.sync_copy(off_hbm.at[bg, :, pl.ds(t, CH)], off_buf)
        pltpu.sync_copy(qs_hbm.at[bg, pl.ds(t, CH)], q_buf)

        for p in range(P):
            pi = float(p // 3 - 1)
            pj = float(p % 3 - 1)

            def idx_grp(j, _, p=p, pi=pi, pj=pj):
                lanes = j * 16
                tv = t + lanes + iota
                hu = tv >> 7
                wu = tv & 127
                offy = off_buf[2 * p, pl.ds(lanes, 16)]
                offx = off_buf[2 * p + 1, pl.ds(lanes, 16)]
                cy = (hu.astype(jnp.float32) + 0.5) * 0.5 - 0.5 + pi + offy
                cx = (wu.astype(jnp.float32) + 0.5) * 0.5 - 0.5 + pj + offx
                yt = cy.astype(jnp.int32)
                y0 = jnp.where(yt.astype(jnp.float32) > cy, yt - 1, yt)
                xt = cx.astype(jnp.int32)
                x0 = jnp.where(xt.astype(jnp.float32) > cx, xt - 1, xt)
                wy = cy - y0.astype(jnp.float32)
                wx = cx - x0.astype(jnp.float32)
                for n, (dy, dx) in enumerate(((0, 0), (0, 1), (1, 0), (1, 1))):
                    yi = y0 + dy
                    xi = x0 + dx
                    ok = (yi >= 0) & (yi <= H - 1) & (xi >= 0) & (xi <= W - 1)
                    wn = (wy if dy else (1.0 - wy)) * (wx if dx else (1.0 - wx))
                    wn = jnp.where(ok, wn, 0.0)
                    yc = jnp.clip(yi, 0, H - 1)
                    xc = jnp.clip(xi, 0, W - 1)
                    idx_buf[p, n, pl.ds(lanes, 16)] = bg * NSITE + yc * W + xc
                    w_buf[p, n, pl.ds(lanes, 16)] = wn
                return 0

            lax.fori_loop(0, CH // 16, idx_grp, 0)

        def issue(tab, p, buf, sem):
            return [
                pltpu.async_copy(tab.at[idx_buf.at[p, n]],
                                 buf.at[pl.ds(n * CH, CH)], sem)
                for n in range(4)
            ]

        lane15 = iota == 15

        descs = issue(kt_hbm, 0, kbufs[0], sems[0])
        for p in range(P):
            nxt = (issue(kt_hbm, p + 1, kbufs[(p + 1) % 2], sems[(p + 1) % 2])
                   if p + 1 < P else None)
            for d in descs:
                d.wait()
            descs = nxt
            krows = kbufs[p % 2]

            def dot_smp(s2, _, p=p, krows=krows):
                for u in range(2):
                    s = s2 * 2 + u
                    w0 = w_buf[p, 0, pl.ds(s, 16)][0]
                    w1 = w_buf[p, 1, pl.ds(s, 16)][0]
                    w2 = w_buf[p, 2, pl.ds(s, 16)][0]
                    w3 = w_buf[p, 3, pl.ds(s, 16)][0]
                    acc0 = (w0 * krows[s, pl.ds(0, 16)]
                            + w1 * krows[CH + s, pl.ds(0, 16)]
                            + w2 * krows[2 * CH + s, pl.ds(0, 16)]
                            + w3 * krows[3 * CH + s, pl.ds(0, 16)])
                    acc1 = (w0 * krows[s, pl.ds(16, 16)]
                            + w1 * krows[CH + s, pl.ds(16, 16)]
                            + w2 * krows[2 * CH + s, pl.ds(16, 16)]
                            + w3 * krows[3 * CH + s, pl.ds(16, 16)])
                    prod = (acc0 * q_buf[s, pl.ds(0, 16)]
                            + acc1 * q_buf[s, pl.ds(16, 16)])
                    csum = plsc.cumsum(prod)
                    plsc.store_scatter(attn.at[p],
                                       [jnp.full((16,), s, jnp.int32)],
                                       csum, mask=lane15)
                return 0

            lax.fori_loop(0, CH // 2, dot_smp, 0)

        def smax_grp(j, _):
            lanes = j * 16
            ls = [attn[p, pl.ds(lanes, 16)] for p in range(P)]
            m = ls[0]
            for p in range(1, P):
                m = jnp.maximum(m, ls[p])
            es = [jnp.exp(l - m) for l in ls]
            ssum = es[0]
            for p in range(1, P):
                ssum = ssum + es[p]
            inv = 1.0 / ssum
            for p in range(P):
                a = es[p] * inv
                for n in range(4):
                    w_buf[p, n, pl.ds(lanes, 16)] = (
                        a * w_buf[p, n, pl.ds(lanes, 16)])
            return 0

        lax.fori_loop(0, CH // 16, smax_grp, 0)

        descs = issue(vt_hbm, 0, vbufs[0], sems[0])
        for p in range(P):
            nxt = (issue(vt_hbm, p + 1, vbufs[(p + 1) % 2], sems[(p + 1) % 2])
                   if p + 1 < P else None)
            for d in descs:
                d.wait()
            descs = nxt
            vrows = vbufs[p % 2]

            def v_smp(s2, _, p=p, vrows=vrows):
                for u in range(2):
                    s = s2 * 2 + u
                    w0 = w_buf[p, 0, pl.ds(s, 16)][0]
                    w1 = w_buf[p, 1, pl.ds(s, 16)][0]
                    w2 = w_buf[p, 2, pl.ds(s, 16)][0]
                    w3 = w_buf[p, 3, pl.ds(s, 16)][0]
                    for h in range(DG // 16):
                        sl = pl.ds(h * 16, 16)
                        v = (w0 * vrows[s, sl]
                             + w1 * vrows[CH + s, sl]
                             + w2 * vrows[2 * CH + s, sl]
                             + w3 * vrows[3 * CH + s, sl])
                        if p == 0:
                            out_buf[s, sl] = v
                        else:
                            out_buf[s, sl] = out_buf[s, sl] + v
                return 0

            lax.fori_loop(0, CH // 2, v_smp, 0)

        pltpu.sync_copy(out_buf, out_hbm.at[bg, pl.ds(t, CH)])
        return 0

    lax.fori_loop(0, NCHUNK, chunk_body, 0)


_sc_call = functools.partial(
    pl.kernel,
    out_type=jax.ShapeDtypeStruct((2 * G, NPIX, DG), jnp.float32),
    mesh=plsc.VectorSubcoreMesh(core_axis_name="c", subcore_axis_name="s"),
    compiler_params=pltpu.CompilerParams(
        needs_layout_passes=False, use_tc_tiling_on_sc=False
    ),
    scratch_types=[
        pltpu.VMEM((2 * P, CH), jnp.float32),
        pltpu.VMEM((CH, E), jnp.float32),
        pltpu.VMEM((P, 4, CH), jnp.int32),
        pltpu.VMEM((P, 4, CH + 16), jnp.float32),
        pltpu.VMEM((4 * CH, E), jnp.float32),
        pltpu.VMEM((4 * CH, E), jnp.float32),
        pltpu.VMEM((4 * CH, DG), jnp.float32),
        pltpu.VMEM((4 * CH, DG), jnp.float32),
        pltpu.VMEM((P, CH + 16), jnp.float32),
        pltpu.VMEM((CH, DG), jnp.float32),
        pltpu.SemaphoreType.DMA,
        pltpu.SemaphoreType.DMA,
    ],
)(_sc_body)


def kernel(y, x, Wq, bq, Wk, bk, Woff, boff):
    B = y.shape[0]
    y2 = y.reshape(B, y.shape[1], NPIX)
    x2 = x.reshape(B, x.shape[1], NSITE)

    q = _conv_tc(y2, Wq, bq, 2048)
    Wkx = jnp.concatenate([Wk, Woff], axis=0)
    bkx = jnp.concatenate([bk, boff], axis=0)
    kx = _conv_tc(x2, Wkx, bkx, 2048)

    k_nat = kx[:, :G * E]
    off_raw = kx[:, G * E:]

    qs = q.reshape(B, G, E, NPIX).transpose(0, 1, 3, 2).reshape(B * G, NPIX, E)
    kt = k_nat.reshape(B, G, E, NSITE).transpose(0, 1, 3, 2).reshape(B * G * NSITE, E)
    vt = x2.reshape(B, G, DG, NSITE).transpose(0, 1, 3, 2).reshape(B * G * NSITE, DG)
    t_ = off_raw.reshape(B, G * P * 2, UP, UP, H, W)
    t_ = t_.transpose(0, 1, 4, 2, 5, 3).reshape(B, G * P * 2, HU, WU)
    off_s = t_.reshape(B * G, P * 2, NPIX)

    out = _sc_call(qs, off_s, kt, vt)
    return out.transpose(0, 2, 1).reshape(B, G * DG, HU, WU)


# speedup vs baseline: 8182.5336x; 1.0117x over previous
import functools

import jax
import jax.numpy as jnp
from jax import lax
from jax.experimental import pallas as pl
from jax.experimental.pallas import tpu as pltpu
from jax.experimental.pallas import tpu_sc as plsc

UP = 2
P = 9
G = 4
E = 32
DG = 64
H = 64
W = 64
HU = H * UP
WU = W * UP
NPIX = HU * WU
NSITE = H * W

NC = 2
NS = 16
NW = NC * NS
CH = 128
PIX_PER_W = 2 * G * NPIX // NW
NCHUNK = PIX_PER_W // CH


def _mm_body(x_ref, w_ref, b_ref, o_ref):
    o_ref[...] = (
        jnp.dot(w_ref[...], x_ref[...], preferred_element_type=jnp.float32)
        + b_ref[...]
    )


def _conv_tc(x2, Wc, bc, tn):
    B, C, S = x2.shape
    OC = Wc.shape[0]
    grid = (B, S // tn)
    return pl.pallas_call(
        _mm_body,
        grid=grid,
        in_specs=[
            pl.BlockSpec((None, C, tn), lambda b, t: (b, 0, t)),
            pl.BlockSpec((OC, C), lambda b, t: (0, 0)),
            pl.BlockSpec((OC, 1), lambda b, t: (0, 0)),
        ],
        out_specs=pl.BlockSpec((None, OC, tn), lambda b, t: (b, 0, t)),
        out_shape=jax.ShapeDtypeStruct((B, OC, S), jnp.float32),
    )(x2, Wc, bc.reshape(OC, 1))


def _sc_body(qs_hbm, off_hbm, kt_hbm, vt_hbm, out_hbm,
             off_buf, q_buf, idx_buf, w_buf, krows_a, krows_b, vrows_a, vrows_b,
             attn, out_buf, sem_a, sem_b, sem_c, sem_d):
    wid = lax.axis_index("s") * NC + lax.axis_index("c")
    bg = wid // 4
    part = wid - bg * 4
    t0 = part * PIX_PER_W
    iota = lax.iota(jnp.int32, 16)
    kbufs = (krows_a, krows_b)
    vbufs = (vrows_a, vrows_b)
    ksems = (sem_a, sem_b)
    vsems = (sem_c, sem_d)

    def chunk_body(ci, _):
        t = t0 + ci * CH
        pltpu.sync_copy(off_hbm.at[bg, :, pl.ds(t, CH)], off_buf)
        pltpu.sync_copy(qs_hbm.at[bg, pl.ds(t, CH)], q_buf)

        for p in range(P):
            pi = float(p // 3 - 1)
            pj = float(p % 3 - 1)

            def idx_grp(j, _, p=p, pi=pi, pj=pj):
                lanes = j * 16
                tv = t + lanes + iota
                hu = tv >> 7
                wu = tv & 127
                offy = off_buf[2 * p, pl.ds(lanes, 16)]
                offx = off_buf[2 * p + 1, pl.ds(lanes, 16)]
                cy = (hu.astype(jnp.float32) + 0.5) * 0.5 - 0.5 + pi + offy
                cx = (wu.astype(jnp.float32) + 0.5) * 0.5 - 0.5 + pj + offx
                yt = cy.astype(jnp.int32)
                y0 = jnp.where(yt.astype(jnp.float32) > cy, yt - 1, yt)
                xt = cx.astype(jnp.int32)
                x0 = jnp.where(xt.astype(jnp.float32) > cx, xt - 1, xt)
                wy = cy - y0.astype(jnp.float32)
                wx = cx - x0.astype(jnp.float32)
                for n, (dy, dx) in enumerate(((0, 0), (0, 1), (1, 0), (1, 1))):
                    yi = y0 + dy
                    xi = x0 + dx
                    ok = (yi >= 0) & (yi <= H - 1) & (xi >= 0) & (xi <= W - 1)
                    wn = (wy if dy else (1.0 - wy)) * (wx if dx else (1.0 - wx))
                    wn = jnp.where(ok, wn, 0.0)
                    yc = jnp.clip(yi, 0, H - 1)
                    xc = jnp.clip(xi, 0, W - 1)
                    idx_buf[p, n, pl.ds(lanes, 16)] = bg * NSITE + yc * W + xc
                    w_buf[p, n, pl.ds(lanes, 16)] = wn
                return 0

            lax.fori_loop(0, CH // 16, idx_grp, 0)

        def issue(tab, p, buf, sem):
            return [
                pltpu.async_copy(tab.at[idx_buf.at[p, n]],
                                 buf.at[pl.ds(n * CH, CH)], sem)
                for n in range(4)
            ]

        lane15 = iota == 15

        kd = [issue(kt_hbm, 0, kbufs[0], ksems[0]),
              issue(kt_hbm, 1, kbufs[1], ksems[1])]
        vd = [issue(vt_hbm, 0, vbufs[0], vsems[0]),
              issue(vt_hbm, 1, vbufs[1], vsems[1])]
        for p in range(P):
            for d in kd[p % 2]:
                d.wait()
            krows = kbufs[p % 2]

            def dot_smp(s2, _, p=p, krows=krows):
                for u in range(2):
                    s = s2 * 2 + u
                    w0 = w_buf[p, 0, pl.ds(s, 16)][0]
                    w1 = w_buf[p, 1, pl.ds(s, 16)][0]
                    w2 = w_buf[p, 2, pl.ds(s, 16)][0]
                    w3 = w_buf[p, 3, pl.ds(s, 16)][0]
                    acc0 = (w0 * krows[s, pl.ds(0, 16)]
                            + w1 * krows[CH + s, pl.ds(0, 16)]
                            + w2 * krows[2 * CH + s, pl.ds(0, 16)]
                            + w3 * krows[3 * CH + s, pl.ds(0, 16)])
                    acc1 = (w0 * krows[s, pl.ds(16, 16)]
                            + w1 * krows[CH + s, pl.ds(16, 16)]
                            + w2 * krows[2 * CH + s, pl.ds(16, 16)]
                            + w3 * krows[3 * CH + s, pl.ds(16, 16)])
                    prod = (acc0 * q_buf[s, pl.ds(0, 16)]
                            + acc1 * q_buf[s, pl.ds(16, 16)])
                    csum = plsc.cumsum(prod)
                    plsc.store_scatter(attn.at[p],
                                       [jnp.full((16,), s, jnp.int32)],
                                       csum, mask=lane15)
                return 0

            lax.fori_loop(0, CH // 2, dot_smp, 0)
            if p + 2 < P:
                kd[p % 2] = issue(kt_hbm, p + 2, kbufs[p % 2], ksems[p % 2])

        def smax_grp(j, _):
            lanes = j * 16
            ls = [attn[p, pl.ds(lanes, 16)] for p in range(P)]
            m = ls[0]
            for p in range(1, P):
                m = jnp.maximum(m, ls[p])
            es = [jnp.exp(l - m) for l in ls]
            ssum = es[0]
            for p in range(1, P):
                ssum = ssum + es[p]
            inv = 1.0 / ssum
            for p in range(P):
                a = es[p] * inv
                for n in range(4):
                    w_buf[p, n, pl.ds(lanes, 16)] = (
                        a * w_buf[p, n, pl.ds(lanes, 16)])
            return 0

        lax.fori_loop(0, CH // 16, smax_grp, 0)

        for p in range(P):
            for d in vd[p % 2]:
                d.wait()
            vrows = vbufs[p % 2]

            def v_smp(s2, _, p=p, vrows=vrows):
                for u in range(2):
                    s = s2 * 2 + u
                    w0 = w_buf[p, 0, pl.ds(s, 16)][0]
                    w1 = w_buf[p, 1, pl.ds(s, 16)][0]
                    w2 = w_buf[p, 2, pl.ds(s, 16)][0]
                    w3 = w_buf[p, 3, pl.ds(s, 16)][0]
                    for h in range(DG // 16):
                        sl = pl.ds(h * 16, 16)
                        v = (w0 * vrows[s, sl]
                             + w1 * vrows[CH + s, sl]
                             + w2 * vrows[2 * CH + s, sl]
                             + w3 * vrows[3 * CH + s, sl])
                        if p == 0:
                            out_buf[s, sl] = v
                        else:
                            out_buf[s, sl] = out_buf[s, sl] + v
                return 0

            lax.fori_loop(0, CH // 2, v_smp, 0)
            if p + 2 < P:
                vd[p % 2] = issue(vt_hbm, p + 2, vbufs[p % 2], vsems[p % 2])

        pltpu.sync_copy(out_buf, out_hbm.at[bg, pl.ds(t, CH)])
        return 0

    lax.fori_loop(0, NCHUNK, chunk_body, 0)


_sc_call = functools.partial(
    pl.kernel,
    out_type=jax.ShapeDtypeStruct((2 * G, NPIX, DG), jnp.float32),
    mesh=plsc.VectorSubcoreMesh(core_axis_name="c", subcore_axis_name="s"),
    compiler_params=pltpu.CompilerParams(
        needs_layout_passes=False, use_tc_tiling_on_sc=False
    ),
    scratch_types=[
        pltpu.VMEM((2 * P, CH), jnp.float32),
        pltpu.VMEM((CH, E), jnp.float32),
        pltpu.VMEM((P, 4, CH), jnp.int32),
        pltpu.VMEM((P, 4, CH + 16), jnp.float32),
        pltpu.VMEM((4 * CH, E), jnp.float32),
        pltpu.VMEM((4 * CH, E), jnp.float32),
        pltpu.VMEM((4 * CH, DG), jnp.float32),
        pltpu.VMEM((4 * CH, DG), jnp.float32),
        pltpu.VMEM((P, CH + 16), jnp.float32),
        pltpu.VMEM((CH, DG), jnp.float32),
        pltpu.SemaphoreType.DMA,
        pltpu.SemaphoreType.DMA,
        pltpu.SemaphoreType.DMA,
        pltpu.SemaphoreType.DMA,
    ],
)(_sc_body)


def kernel(y, x, Wq, bq, Wk, bk, Woff, boff):
    B = y.shape[0]
    y2 = y.reshape(B, y.shape[1], NPIX)
    x2 = x.reshape(B, x.shape[1], NSITE)

    q = _conv_tc(y2, Wq, bq, 2048)
    Wkx = jnp.concatenate([Wk, Woff], axis=0)
    bkx = jnp.concatenate([bk, boff], axis=0)
    kx = _conv_tc(x2, Wkx, bkx, 2048)

    k_nat = kx[:, :G * E]
    off_raw = kx[:, G * E:]

    qs = q.reshape(B, G, E, NPIX).transpose(0, 1, 3, 2).reshape(B * G, NPIX, E)
    kt = k_nat.reshape(B, G, E, NSITE).transpose(0, 1, 3, 2).reshape(B * G * NSITE, E)
    vt = x2.reshape(B, G, DG, NSITE).transpose(0, 1, 3, 2).reshape(B * G * NSITE, DG)
    t_ = off_raw.reshape(B, G * P * 2, UP, UP, H, W)
    t_ = t_.transpose(0, 1, 4, 2, 5, 3).reshape(B, G * P * 2, HU, WU)
    off_s = t_.reshape(B * G, P * 2, NPIX)

    out = _sc_call(qs, off_s, kt, vt)
    return out.transpose(0, 2, 1).reshape(B, G * DG, HU, WU)


# speedup vs baseline: 8213.2292x; 1.0038x over previous
import functools

import jax
import jax.numpy as jnp
from jax import lax
from jax.experimental import pallas as pl
from jax.experimental.pallas import tpu as pltpu
from jax.experimental.pallas import tpu_sc as plsc

UP = 2
P = 9
G = 4
E = 32
DG = 64
H = 64
W = 64
HU = H * UP
WU = W * UP
NPIX = HU * WU
NSITE = H * W

NC = 2
NS = 16
NW = NC * NS
CH = 128
PIX_PER_W = 2 * G * NPIX // NW
NCHUNK = PIX_PER_W // CH


def _mm_body(x_ref, w_ref, b_ref, o_ref):
    o_ref[...] = (
        jnp.dot(w_ref[...], x_ref[...], preferred_element_type=jnp.float32)
        + b_ref[...]
    )


def _conv_tc(x2, Wc, bc, tn):
    B, C, S = x2.shape
    OC = Wc.shape[0]
    grid = (B, S // tn)
    return pl.pallas_call(
        _mm_body,
        grid=grid,
        in_specs=[
            pl.BlockSpec((None, C, tn), lambda b, t: (b, 0, t)),
            pl.BlockSpec((OC, C), lambda b, t: (0, 0)),
            pl.BlockSpec((OC, 1), lambda b, t: (0, 0)),
        ],
        out_specs=pl.BlockSpec((None, OC, tn), lambda b, t: (b, 0, t)),
        out_shape=jax.ShapeDtypeStruct((B, OC, S), jnp.float32),
    )(x2, Wc, bc.reshape(OC, 1))


def _sc_body(qs_hbm, off_hbm, kt_hbm, vt_hbm, out_hbm,
             off_buf, q_buf, idx_buf, w_buf, krows_a, krows_b, vrows_a, vrows_b,
             attn, out_buf, sem_a, sem_b, sem_c, sem_d):
    wid = lax.axis_index("s") * NC + lax.axis_index("c")
    bg = wid // 4
    part = wid - bg * 4
    t0 = part * PIX_PER_W
    iota = lax.iota(jnp.int32, 16)
    kbufs = (krows_a, krows_b)
    vbufs = (vrows_a, vrows_b)
    ksems = (sem_a, sem_b)
    vsems = (sem_c, sem_d)

    def chunk_body(ci, _):
        t = t0 + ci * CH
        pltpu.sync_copy(off_hbm.at[bg, :, pl.ds(t, CH)], off_buf)
        pltpu.sync_copy(qs_hbm.at[bg, pl.ds(t, CH)], q_buf)

        for p in range(P):
            pi = float(p // 3 - 1)
            pj = float(p % 3 - 1)

            def idx_grp(j, _, p=p, pi=pi, pj=pj):
                lanes = j * 16
                tv = t + lanes + iota
                hu = tv >> 7
                wu = tv & 127
                offy = off_buf[2 * p, pl.ds(lanes, 16)]
                offx = off_buf[2 * p + 1, pl.ds(lanes, 16)]
                cy = (hu.astype(jnp.float32) + 0.5) * 0.5 - 0.5 + pi + offy
                cx = (wu.astype(jnp.float32) + 0.5) * 0.5 - 0.5 + pj + offx
                yt = cy.astype(jnp.int32)
                y0 = jnp.where(yt.astype(jnp.float32) > cy, yt - 1, yt)
                xt = cx.astype(jnp.int32)
                x0 = jnp.where(xt.astype(jnp.float32) > cx, xt - 1, xt)
                wy = cy - y0.astype(jnp.float32)
                wx = cx - x0.astype(jnp.float32)
                for n, (dy, dx) in enumerate(((0, 0), (0, 1), (1, 0), (1, 1))):
                    yi = y0 + dy
                    xi = x0 + dx
                    ok = (yi >= 0) & (yi <= H - 1) & (xi >= 0) & (xi <= W - 1)
                    wn = (wy if dy else (1.0 - wy)) * (wx if dx else (1.0 - wx))
                    wn = jnp.where(ok, wn, 0.0)
                    yc = jnp.clip(yi, 0, H - 1)
                    xc = jnp.clip(xi, 0, W - 1)
                    idx_buf[p, n, pl.ds(lanes, 16)] = bg * NSITE + yc * W + xc
                    w_buf[p, n, pl.ds(lanes, 16)] = wn
                return 0

            lax.fori_loop(0, CH // 16, idx_grp, 0)

        def issue(tab, p, buf, sem):
            return [
                pltpu.async_copy(tab.at[idx_buf.at[p, n]],
                                 buf.at[pl.ds(n * CH, CH)], sem)
                for n in range(4)
            ]

        lane15 = iota == 15

        kd = [issue(kt_hbm, 0, kbufs[0], ksems[0]),
              issue(kt_hbm, 1, kbufs[1], ksems[1])]
        vd = [issue(vt_hbm, 0, vbufs[0], vsems[0]),
              issue(vt_hbm, 1, vbufs[1], vsems[1])]
        for p in range(P):
            for d in kd[p % 2]:
                d.wait()
            krows = kbufs[p % 2]

            def dot_smp(s2, _, p=p, krows=krows):
                for u in range(4):
                    s = s2 * 4 + u
                    w0 = w_buf[p, 0, pl.ds(s, 16)][0]
                    w1 = w_buf[p, 1, pl.ds(s, 16)][0]
                    w2 = w_buf[p, 2, pl.ds(s, 16)][0]
                    w3 = w_buf[p, 3, pl.ds(s, 16)][0]
                    acc0 = (w0 * krows[s, pl.ds(0, 16)]
                            + w1 * krows[CH + s, pl.ds(0, 16)]
                            + w2 * krows[2 * CH + s, pl.ds(0, 16)]
                            + w3 * krows[3 * CH + s, pl.ds(0, 16)])
                    acc1 = (w0 * krows[s, pl.ds(16, 16)]
                            + w1 * krows[CH + s, pl.ds(16, 16)]
                            + w2 * krows[2 * CH + s, pl.ds(16, 16)]
                            + w3 * krows[3 * CH + s, pl.ds(16, 16)])
                    prod = (acc0 * q_buf[s, pl.ds(0, 16)]
                            + acc1 * q_buf[s, pl.ds(16, 16)])
                    csum = plsc.cumsum(prod)
                    plsc.store_scatter(attn.at[p],
                                       [jnp.full((16,), s, jnp.int32)],
                                       csum, mask=lane15)
                return 0

            lax.fori_loop(0, CH // 4, dot_smp, 0)
            if p + 2 < P:
                kd[p % 2] = issue(kt_hbm, p + 2, kbufs[p % 2], ksems[p % 2])

        def smax_grp(j, _):
            lanes = j * 16
            ls = [attn[p, pl.ds(lanes, 16)] for p in range(P)]
            m = ls[0]
            for p in range(1, P):
                m = jnp.maximum(m, ls[p])
            es = [jnp.exp(l - m) for l in ls]
            ssum = es[0]
            for p in range(1, P):
                ssum = ssum + es[p]
            inv = 1.0 / ssum
            for p in range(P):
                a = es[p] * inv
                for n in range(4):
                    w_buf[p, n, pl.ds(lanes, 16)] = (
                        a * w_buf[p, n, pl.ds(lanes, 16)])
            return 0

        lax.fori_loop(0, CH // 16, smax_grp, 0)

        for p in range(P):
            for d in vd[p % 2]:
                d.wait()
            vrows = vbufs[p % 2]

            def v_smp(s2, _, p=p, vrows=vrows):
                for u in range(2):
                    s = s2 * 2 + u
                    w0 = w_buf[p, 0, pl.ds(s, 16)][0]
                    w1 = w_buf[p, 1, pl.ds(s, 16)][0]
                    w2 = w_buf[p, 2, pl.ds(s, 16)][0]
                    w3 = w_buf[p, 3, pl.ds(s, 16)][0]
                    for h in range(DG // 16):
                        sl = pl.ds(h * 16, 16)
                        v = (w0 * vrows[s, sl]
                             + w1 * vrows[CH + s, sl]
                             + w2 * vrows[2 * CH + s, sl]
                             + w3 * vrows[3 * CH + s, sl])
                        if p == 0:
                            out_buf[s, sl] = v
                        else:
                            out_buf[s, sl] = out_buf[s, sl] + v
                return 0

            lax.fori_loop(0, CH // 2, v_smp, 0)
            if p + 2 < P:
                vd[p % 2] = issue(vt_hbm, p + 2, vbufs[p % 2], vsems[p % 2])

        pltpu.sync_copy(out_buf, out_hbm.at[bg, pl.ds(t, CH)])
        return 0

    lax.fori_loop(0, NCHUNK, chunk_body, 0)


_sc_call = functools.partial(
    pl.kernel,
    out_type=jax.ShapeDtypeStruct((2 * G, NPIX, DG), jnp.float32),
    mesh=plsc.VectorSubcoreMesh(core_axis_name="c", subcore_axis_name="s"),
    compiler_params=pltpu.CompilerParams(
        needs_layout_passes=False, use_tc_tiling_on_sc=False
    ),
    scratch_types=[
        pltpu.VMEM((2 * P, CH), jnp.float32),
        pltpu.VMEM((CH, E), jnp.float32),
        pltpu.VMEM((P, 4, CH), jnp.int32),
        pltpu.VMEM((P, 4, CH + 16), jnp.float32),
        pltpu.VMEM((4 * CH, E), jnp.float32),
        pltpu.VMEM((4 * CH, E), jnp.float32),
        pltpu.VMEM((4 * CH, DG), jnp.float32),
        pltpu.VMEM((4 * CH, DG), jnp.float32),
        pltpu.VMEM((P, CH + 16), jnp.float32),
        pltpu.VMEM((CH, DG), jnp.float32),
        pltpu.SemaphoreType.DMA,
        pltpu.SemaphoreType.DMA,
        pltpu.SemaphoreType.DMA,
        pltpu.SemaphoreType.DMA,
    ],
)(_sc_body)


def kernel(y, x, Wq, bq, Wk, bk, Woff, boff):
    B = y.shape[0]
    y2 = y.reshape(B, y.shape[1], NPIX)
    x2 = x.reshape(B, x.shape[1], NSITE)

    q = _conv_tc(y2, Wq, bq, 2048)
    Wkx = jnp.concatenate([Wk, Woff], axis=0)
    bkx = jnp.concatenate([bk, boff], axis=0)
    kx = _conv_tc(x2, Wkx, bkx, 2048)

    k_nat = kx[:, :G * E]
    off_raw = kx[:, G * E:]

    qs = q.reshape(B, G, E, NPIX).transpose(0, 1, 3, 2).reshape(B * G, NPIX, E)
    kt = k_nat.reshape(B, G, E, NSITE).transpose(0, 1, 3, 2).reshape(B * G * NSITE, E)
    vt = x2.reshape(B, G, DG, NSITE).transpose(0, 1, 3, 2).reshape(B * G * NSITE, DG)
    t_ = off_raw.reshape(B, G * P * 2, UP, UP, H, W)
    t_ = t_.transpose(0, 1, 4, 2, 5, 3).reshape(B, G * P * 2, HU, WU)
    off_s = t_.reshape(B * G, P * 2, NPIX)

    out = _sc_call(qs, off_s, kt, vt)
    return out.transpose(0, 2, 1).reshape(B, G * DG, HU, WU)
